# baseline (speedup 1.0000x reference)
import jax
import jax.numpy as jnp
from jax import lax
from jax.experimental import pallas as pl
from jax.experimental.pallas import tpu as pltpu
from jax.experimental.pallas import tpu_sc as plsc

N_PHYS = 10000
N_EDGES = 320000
N_LATENT = 4096
D_IN = 128
D_LIFT = 16
HID = 64

NC = 2
NS = 16
LANES = 16
CH = 128
ROWS_PER_TILE = N_LATENT // NS
BE = 6400
NHALF = 5
HE = N_EDGES // NHALF
QC = (HE // NC) // CH
NJ_BASE = QC // NS
EXTRA = QC - NJ_BASE * NS


def _lift_body(feat_ref, w_ref, b_ref, out_ref):
    out_ref[...] = feat_ref[...] @ w_ref[...] + b_ref[...]


def _lift(phys_feat, W_lift, b_lift):
    return pl.pallas_call(
        _lift_body,
        out_shape=jax.ShapeDtypeStruct((N_PHYS, D_LIFT), jnp.float32),
    )(phys_feat, W_lift, b_lift.reshape(1, D_LIFT))


def _gather_body(lat_hbm, phys_hbm, lifted_hbm, src_hbm, dst_hbm,
                 kin_hbm, lsrc_hbm,
                 lift_sh, lat_v, phys_v, src_v, dst_v, rows_v, kin_v,
                 sem_in, sem_g, sem_out):
    c = lax.axis_index("c")
    s = lax.axis_index("s")

    @pl.when(s == 0)
    def _():
        pltpu.sync_copy(lifted_hbm, lift_sh)

    nj = jnp.where(s < EXTRA, NJ_BASE + 1, NJ_BASE)
    core_base = c * (QC * CH)

    def stage(j, slot):
        eb = core_base + (s + NS * j) * CH
        pltpu.async_copy(src_hbm.at[pl.ds(eb, CH)], src_v.at[slot], sem_in)
        pltpu.async_copy(dst_hbm.at[pl.ds(eb, CH)], dst_v.at[slot], sem_in)

    stage(0, 0)
    pltpu.sync_copy(lat_hbm, lat_v)
    pltpu.sync_copy(phys_hbm, phys_v)
    plsc.subcore_barrier()

    z16i = jnp.zeros((LANES,), jnp.int32)
    o16i = jnp.ones((LANES,), jnp.int32)

    def lb(j, _):
        slot = lax.rem(j, 3)
        nxt = lax.rem(j + 1, 3)
        rslot = lax.rem(j, 2)
        eb = core_base + (s + NS * j) * CH

        @pl.when(j >= 2)
        def _():
            pltpu.make_async_copy(rows_v.at[0], lsrc_hbm.at[pl.ds(0, CH)],
                                  sem_out).wait()
            pltpu.make_async_copy(kin_v.at[0], kin_hbm.at[:, pl.ds(0, CH)],
                                  sem_out).wait()

        pltpu.make_async_copy(src_hbm.at[pl.ds(0, CH)], src_v.at[slot],
                              sem_in).wait()
        pltpu.make_async_copy(dst_hbm.at[pl.ds(0, CH)], dst_v.at[slot],
                              sem_in).wait()
        g = pltpu.async_copy(lift_sh.at[src_v.at[slot]], rows_v.at[rslot],
                             sem_g)
        stage(jnp.minimum(j + 1, nj - 1), nxt)

        for gi in range(CH // LANES):
            sl = pl.ds(gi * LANES, LANES)
            dv = dst_v[slot, sl]
            sv = src_v[slot, sl]
            kin_v[rslot, 0, sl] = plsc.load_gather(lat_v, [dv, z16i])
            kin_v[rslot, 1, sl] = plsc.load_gather(lat_v, [dv, o16i])
            kin_v[rslot, 2, sl] = plsc.load_gather(phys_v, [sv, z16i])
            kin_v[rslot, 3, sl] = plsc.load_gather(phys_v, [sv, o16i])

        g.wait()
        pltpu.async_copy(rows_v.at[rslot], lsrc_hbm.at[pl.ds(eb, CH)],
                         sem_out)
        pltpu.async_copy(kin_v.at[rslot], kin_hbm.at[:, pl.ds(eb, CH)],
                         sem_out)
        return 0

    lax.fori_loop(0, nj, lb, 0)
    pltpu.make_async_copy(src_hbm.at[pl.ds(0, CH)], src_v.at[0],
                          sem_in).wait()
    pltpu.make_async_copy(dst_hbm.at[pl.ds(0, CH)], dst_v.at[0],
                          sem_in).wait()
    for _ in range(2):
        pltpu.make_async_copy(rows_v.at[0], lsrc_hbm.at[pl.ds(0, CH)],
                              sem_out).wait()
        pltpu.make_async_copy(kin_v.at[0], kin_hbm.at[:, pl.ds(0, CH)],
                              sem_out).wait()


def _gather(latent_pos, phys_pos, lifted, edge_src, edge_dst):
    mesh = plsc.VectorSubcoreMesh(core_axis_name="c", subcore_axis_name="s",
                                  num_cores=NC, num_subcores=NS)
    return pl.kernel(
        _gather_body,
        out_type=(jax.ShapeDtypeStruct((4, HE), jnp.float32),
                  jax.ShapeDtypeStruct((HE, D_LIFT), jnp.float32)),
        mesh=mesh,
        compiler_params=pltpu.CompilerParams(
            needs_layout_passes=False, use_tc_tiling_on_sc=False),
        scratch_types=[
            pltpu.VMEM_SHARED((N_PHYS, D_LIFT), jnp.float32),
            pltpu.VMEM((N_LATENT, 2), jnp.float32),
            pltpu.VMEM((N_PHYS, 2), jnp.float32),
            pltpu.VMEM((3, CH), jnp.int32),
            pltpu.VMEM((3, CH), jnp.int32),
            pltpu.VMEM((2, CH, D_LIFT), jnp.float32),
            pltpu.VMEM((2, 4, CH), jnp.float32),
            pltpu.SemaphoreType.DMA,
            pltpu.SemaphoreType.DMA,
            pltpu.SemaphoreType.DMA,
        ],
    )(latent_pos, phys_pos, lifted, edge_src, edge_dst)


def _tdot(w, x):
    return lax.dot_general(w.astype(jnp.bfloat16), x.astype(jnp.bfloat16),
                           (((0,), (0,)), ((), ())),
                           preferred_element_type=jnp.float32)


def _mlp_body(kin_ref, w0, b0, w1, b1, w2, b2, w3, b3, out_ref):
    x = kin_ref[...]
    h = jax.nn.gelu(_tdot(w0[...], x) + b0[...])
    h = jax.nn.gelu(_tdot(w1[...], h) + b1[...])
    h = jax.nn.gelu(_tdot(w2[...], h) + b2[...])
    out_ref[...] = _tdot(w3[...], h) + b3[...]


def _mlp(kin, W0, b0, W1, b1, W2, b2, W3, b3):
    full = lambda shape: pl.BlockSpec(shape, lambda i: (0, 0))
    return pl.pallas_call(
        _mlp_body,
        grid=(HE // BE,),
        in_specs=[
            pl.BlockSpec((4, BE), lambda i: (0, i)),
            full((4, HID)), full((HID, 1)),
            full((HID, HID)), full((HID, 1)),
            full((HID, HID)), full((HID, 1)),
            full((HID, D_LIFT)), full((D_LIFT, 1)),
        ],
        out_specs=pl.BlockSpec((D_LIFT, BE), lambda i: (0, i)),
        out_shape=jax.ShapeDtypeStruct((D_LIFT, HE), jnp.float32),
    )(kin, W0, b0.reshape(HID, 1), W1, b1.reshape(HID, 1),
      W2, b2.reshape(HID, 1), W3, b3.reshape(D_LIFT, 1))


def _scatter_body(kt_hbm, lsrc_hbm, dst_hbm, part_hbm, cntp_hbm,
                  acc_sh, cnt_sh, kt_v, ls_v, msg_v, dst_v, ones_v,
                  zb_v, zc_v, cb_v, crep_v, sem_in, sem_sc):
    c = lax.axis_index("c")
    s = lax.axis_index("s")
    row0 = s * ROWS_PER_TILE
    nj = jnp.where(s < EXTRA, NJ_BASE + 1, NJ_BASE)
    core_base = c * (QC * CH)

    def stage(j, slot):
        eb = core_base + (s + NS * j) * CH
        pltpu.async_copy(kt_hbm.at[:, pl.ds(eb, CH)], kt_v.at[slot], sem_in)
        pltpu.async_copy(lsrc_hbm.at[pl.ds(eb, CH)], ls_v.at[slot], sem_in)
        pltpu.async_copy(dst_hbm.at[pl.ds(eb, CH)], dst_v.at[slot], sem_in)

    stage(0, 0)
    iota16 = lax.iota(jnp.int32, LANES)

    z16 = jnp.zeros((LANES,), jnp.float32)

    def zb(i, _):
        zb_v[i, :] = z16
        return 0
    lax.fori_loop(0, ROWS_PER_TILE, zb, 0)

    def zc(i, _):
        zc_v[pl.ds(i * LANES, LANES)] = z16
        return 0
    lax.fori_loop(0, ROWS_PER_TILE // LANES, zc, 0)

    def o16(i, _):
        ones_v[pl.ds(i * LANES, LANES)] = jnp.ones((LANES,), jnp.float32)
        return 0
    lax.fori_loop(0, CH // LANES, o16, 0)

    pltpu.sync_copy(zb_v, acc_sh.at[pl.ds(row0, ROWS_PER_TILE)])
    pltpu.sync_copy(zc_v, cnt_sh.at[pl.ds(row0, ROWS_PER_TILE)])
    plsc.subcore_barrier()

    def lb(j, _):
        slot = lax.rem(j, 3)
        nxt = lax.rem(j + 1, 3)

        @pl.when(j >= 2)
        def _():
            pltpu.make_async_copy(msg_v.at[0], acc_sh.at[pl.ds(0, CH)],
                                  sem_sc).wait()
            pltpu.make_async_copy(ones_v, cnt_sh.at[pl.ds(0, CH)],
                                  sem_sc).wait()

        pltpu.make_async_copy(kt_hbm.at[:, pl.ds(0, CH)], kt_v.at[slot],
                              sem_in).wait()
        pltpu.make_async_copy(lsrc_hbm.at[pl.ds(0, CH)], ls_v.at[slot],
                              sem_in).wait()
        pltpu.make_async_copy(dst_hbm.at[pl.ds(0, CH)], dst_v.at[slot],
                              sem_in).wait()
        stage(jnp.minimum(j + 1, nj - 1), nxt)
        sl3 = jnp.full((LANES,), slot, jnp.int32)
        for comp in range(D_LIFT):
            c3 = jnp.full((LANES,), comp, jnp.int32)
            for gi in range(CH // LANES):
                ids = gi * LANES + iota16
                kv = kt_v[slot, comp, pl.ds(gi * LANES, LANES)]
                lg = plsc.load_gather(ls_v, [sl3, ids, c3])
                plsc.store_scatter(msg_v, [sl3, ids, c3], kv * lg)
        pltpu.async_copy(msg_v.at[slot], acc_sh.at[dst_v.at[slot]],
                         sem_sc, add=True)
        pltpu.async_copy(ones_v, cnt_sh.at[dst_v.at[slot]],
                         sem_sc, add=True)
        return 0

    lax.fori_loop(0, nj, lb, 0)
    pltpu.make_async_copy(kt_hbm.at[:, pl.ds(0, CH)], kt_v.at[0],
                          sem_in).wait()
    pltpu.make_async_copy(lsrc_hbm.at[pl.ds(0, CH)], ls_v.at[0],
                          sem_in).wait()
    pltpu.make_async_copy(dst_hbm.at[pl.ds(0, CH)], dst_v.at[0],
                          sem_in).wait()
    for _ in range(2):
        pltpu.make_async_copy(msg_v.at[0], acc_sh.at[pl.ds(0, CH)],
                              sem_sc).wait()
        pltpu.make_async_copy(ones_v, cnt_sh.at[pl.ds(0, CH)],
                              sem_sc).wait()
    plsc.subcore_barrier()

    pltpu.sync_copy(acc_sh.at[pl.ds(row0, ROWS_PER_TILE)],
                    part_hbm.at[c, pl.ds(row0, ROWS_PER_TILE)])
    pltpu.sync_copy(cnt_sh.at[pl.ds(row0, ROWS_PER_TILE)], cb_v)

    def rep(i, _):
        crep_v[pl.ds(i * LANES, LANES)] = plsc.load_gather(
            cb_v, [jnp.full((LANES,), i, jnp.int32)])
        return 0
    lax.fori_loop(0, ROWS_PER_TILE, rep, 0)
    pltpu.sync_copy(crep_v,
                    cntp_hbm.at[c, pl.ds(row0 * D_LIFT,
                                         ROWS_PER_TILE * D_LIFT)])


def _scatter(kt, lsrc, edge_dst):
    mesh = plsc.VectorSubcoreMesh(core_axis_name="c", subcore_axis_name="s",
                                  num_cores=NC, num_subcores=NS)
    return pl.kernel(
        _scatter_body,
        out_type=(jax.ShapeDtypeStruct((NC, N_LATENT, D_LIFT), jnp.float32),
                  jax.ShapeDtypeStruct((NC, N_LATENT * D_LIFT), jnp.float32)),
        mesh=mesh,
        compiler_params=pltpu.CompilerParams(
            needs_layout_passes=False, use_tc_tiling_on_sc=False),
        scratch_types=[
            pltpu.VMEM_SHARED((N_LATENT, D_LIFT), jnp.float32),
            pltpu.VMEM_SHARED((N_LATENT,), jnp.float32),
            pltpu.VMEM((3, D_LIFT, CH), jnp.float32),
            pltpu.VMEM((3, CH, D_LIFT), jnp.float32),
            pltpu.VMEM((3, CH, D_LIFT), jnp.float32),
            pltpu.VMEM((3, CH), jnp.int32),
            pltpu.VMEM((CH,), jnp.float32),
            pltpu.VMEM((ROWS_PER_TILE, D_LIFT), jnp.float32),
            pltpu.VMEM((ROWS_PER_TILE,), jnp.float32),
            pltpu.VMEM((ROWS_PER_TILE,), jnp.float32),
            pltpu.VMEM((ROWS_PER_TILE * D_LIFT,), jnp.float32),
            pltpu.SemaphoreType.DMA,
            pltpu.SemaphoreType.DMA,
        ],
    )(kt, lsrc, edge_dst)


def _combine_body(*refs):
    part_refs = refs[:-1]
    out_ref = refs[-1]
    p = part_refs[0][0] + part_refs[0][1]
    cnt = part_refs[1][0] + part_refs[1][1]
    for i in range(2, len(part_refs), 2):
        p = p + part_refs[i][0] + part_refs[i][1]
        cnt = cnt + part_refs[i + 1][0] + part_refs[i + 1][1]
    out_ref[...] = p / jnp.maximum(cnt, 1.0)


def _combine(*parts):
    return pl.pallas_call(
        _combine_body,
        out_shape=jax.ShapeDtypeStruct((N_LATENT, D_LIFT), jnp.float32),
    )(*parts)


def kernel(phys_pos, phys_feat, batch_idx_phys, edge_src, edge_dst,
           latent_tokens_pos, latent_tokens_batch_idx,
           W_lift, b_lift, W0, b0, W1, b1, W2, b2, W3, b3):
    lifted = _lift(phys_feat, W_lift, b_lift)
    parts = []
    for hidx in range(NHALF):
        sl = slice(hidx * HE, (hidx + 1) * HE)
        kin_h, lsrc_h = _gather(latent_tokens_pos, phys_pos, lifted,
                                edge_src[sl], edge_dst[sl])
        kt_h = _mlp(kin_h, W0, b0, W1, b1, W2, b2, W3, b3)
        part_h, cnt_h = _scatter(kt_h, lsrc_h, edge_dst[sl])
        parts += [part_h, cnt_h.reshape(NC, N_LATENT, D_LIFT)]
    return _combine(*parts)


# speedup vs baseline: 1.2011x; 1.2011x over previous
import jax
import jax.numpy as jnp
from jax import lax
from jax.experimental import pallas as pl
from jax.experimental.pallas import tpu as pltpu
from jax.experimental.pallas import tpu_sc as plsc

N_PHYS = 10000
N_EDGES = 320000
N_LATENT = 4096
D_IN = 128
D_LIFT = 16
HID = 64

NC = 2
NS = 16
LANES = 16
CH = 128
ROWS_PER_TILE = N_LATENT // NS
BE = 6400
NHALF = 2
HE = N_EDGES // NHALF
QC = (HE // NC) // CH
NJ_BASE = QC // NS
EXTRA = QC - NJ_BASE * NS


def _lift_body(feat_ref, w_ref, b_ref, out_ref):
    out_ref[...] = feat_ref[...] @ w_ref[...] + b_ref[...]


def _lift(phys_feat, W_lift, b_lift):
    return pl.pallas_call(
        _lift_body,
        out_shape=jax.ShapeDtypeStruct((N_PHYS, D_LIFT), jnp.float32),
    )(phys_feat, W_lift, b_lift.reshape(1, D_LIFT))


def _gather_body(lat_hbm, phys_hbm, lifted_hbm, src_hbm, dst_hbm,
                 kin_hbm, lsrc_hbm,
                 lift_sh, lat_v, phys_v, src_v, dst_v, rows_v, kin_v,
                 sem_in, sem_g, sem_out):
    c = lax.axis_index("c")
    s = lax.axis_index("s")

    @pl.when(s == 0)
    def _():
        pltpu.sync_copy(lifted_hbm, lift_sh)

    nj = jnp.where(s < EXTRA, NJ_BASE + 1, NJ_BASE)
    core_base = c * (QC * CH)

    def stage(j, slot):
        eb = core_base + (s + NS * j) * CH
        pltpu.async_copy(src_hbm.at[pl.ds(eb, CH)], src_v.at[slot], sem_in)
        pltpu.async_copy(dst_hbm.at[pl.ds(eb, CH)], dst_v.at[slot], sem_in)

    stage(0, 0)
    pltpu.sync_copy(lat_hbm, lat_v)
    pltpu.sync_copy(phys_hbm, phys_v)
    plsc.subcore_barrier()

    z16i = jnp.zeros((LANES,), jnp.int32)
    o16i = jnp.ones((LANES,), jnp.int32)

    def lb(j, _):
        slot = lax.rem(j, 3)
        nxt = lax.rem(j + 1, 3)
        rslot = lax.rem(j, 2)
        eb = core_base + (s + NS * j) * CH

        @pl.when(j >= 2)
        def _():
            pltpu.make_async_copy(rows_v.at[0], lsrc_hbm.at[pl.ds(0, CH)],
                                  sem_out).wait()
            pltpu.make_async_copy(kin_v.at[0], kin_hbm.at[:, pl.ds(0, CH)],
                                  sem_out).wait()

        pltpu.make_async_copy(src_hbm.at[pl.ds(0, CH)], src_v.at[slot],
                              sem_in).wait()
        pltpu.make_async_copy(dst_hbm.at[pl.ds(0, CH)], dst_v.at[slot],
                              sem_in).wait()
        g = pltpu.async_copy(lift_sh.at[src_v.at[slot]], rows_v.at[rslot],
                             sem_g)
        stage(jnp.minimum(j + 1, nj - 1), nxt)

        for gi in range(CH // LANES):
            sl = pl.ds(gi * LANES, LANES)
            dv = dst_v[slot, sl]
            sv = src_v[slot, sl]
            kin_v[rslot, 0, sl] = plsc.load_gather(lat_v, [dv, z16i])
            kin_v[rslot, 1, sl] = plsc.load_gather(lat_v, [dv, o16i])
            kin_v[rslot, 2, sl] = plsc.load_gather(phys_v, [sv, z16i])
            kin_v[rslot, 3, sl] = plsc.load_gather(phys_v, [sv, o16i])

        g.wait()
        pltpu.async_copy(rows_v.at[rslot], lsrc_hbm.at[pl.ds(eb, CH)],
                         sem_out)
        pltpu.async_copy(kin_v.at[rslot], kin_hbm.at[:, pl.ds(eb, CH)],
                         sem_out)
        return 0

    lax.fori_loop(0, nj, lb, 0)
    pltpu.make_async_copy(src_hbm.at[pl.ds(0, CH)], src_v.at[0],
                          sem_in).wait()
    pltpu.make_async_copy(dst_hbm.at[pl.ds(0, CH)], dst_v.at[0],
                          sem_in).wait()
    for _ in range(2):
        pltpu.make_async_copy(rows_v.at[0], lsrc_hbm.at[pl.ds(0, CH)],
                              sem_out).wait()
        pltpu.make_async_copy(kin_v.at[0], kin_hbm.at[:, pl.ds(0, CH)],
                              sem_out).wait()


def _gather(latent_pos, phys_pos, lifted, edge_src, edge_dst):
    mesh = plsc.VectorSubcoreMesh(core_axis_name="c", subcore_axis_name="s",
                                  num_cores=NC, num_subcores=NS)
    return pl.kernel(
        _gather_body,
        out_type=(jax.ShapeDtypeStruct((4, HE), jnp.float32),
                  jax.ShapeDtypeStruct((HE, D_LIFT), jnp.float32)),
        mesh=mesh,
        compiler_params=pltpu.CompilerParams(
            needs_layout_passes=False, use_tc_tiling_on_sc=False),
        scratch_types=[
            pltpu.VMEM_SHARED((N_PHYS, D_LIFT), jnp.float32),
            pltpu.VMEM((N_LATENT, 2), jnp.float32),
            pltpu.VMEM((N_PHYS, 2), jnp.float32),
            pltpu.VMEM((3, CH), jnp.int32),
            pltpu.VMEM((3, CH), jnp.int32),
            pltpu.VMEM((2, CH, D_LIFT), jnp.float32),
            pltpu.VMEM((2, 4, CH), jnp.float32),
            pltpu.SemaphoreType.DMA,
            pltpu.SemaphoreType.DMA,
            pltpu.SemaphoreType.DMA,
        ],
    )(latent_pos, phys_pos, lifted, edge_src, edge_dst)


def _tdot(w, x):
    return lax.dot_general(w.astype(jnp.bfloat16), x.astype(jnp.bfloat16),
                           (((0,), (0,)), ((), ())),
                           preferred_element_type=jnp.float32)


def _mlp_body(kin_ref, w0, b0, w1, b1, w2, b2, w3, b3, out_ref):
    x = kin_ref[...]
    h = jax.nn.gelu(_tdot(w0[...], x) + b0[...])
    h = jax.nn.gelu(_tdot(w1[...], h) + b1[...])
    h = jax.nn.gelu(_tdot(w2[...], h) + b2[...])
    out_ref[...] = _tdot(w3[...], h) + b3[...]


def _mlp(kin, W0, b0, W1, b1, W2, b2, W3, b3):
    full = lambda shape: pl.BlockSpec(shape, lambda i: (0, 0))
    return pl.pallas_call(
        _mlp_body,
        grid=(HE // BE,),
        in_specs=[
            pl.BlockSpec((4, BE), lambda i: (0, i)),
            full((4, HID)), full((HID, 1)),
            full((HID, HID)), full((HID, 1)),
            full((HID, HID)), full((HID, 1)),
            full((HID, D_LIFT)), full((D_LIFT, 1)),
        ],
        out_specs=pl.BlockSpec((D_LIFT, BE), lambda i: (0, i)),
        out_shape=jax.ShapeDtypeStruct((D_LIFT, HE), jnp.float32),
    )(kin, W0, b0.reshape(HID, 1), W1, b1.reshape(HID, 1),
      W2, b2.reshape(HID, 1), W3, b3.reshape(D_LIFT, 1))


def _scatter_body(kt_hbm, lsrc_hbm, dst_hbm, part_hbm, cntp_hbm,
                  acc_sh, cnt_sh, kt_v, ls_v, msg_v, dst_v, ones_v,
                  zb_v, zc_v, cb_v, crep_v, sem_in, sem_sc):
    c = lax.axis_index("c")
    s = lax.axis_index("s")
    row0 = s * ROWS_PER_TILE
    nj = jnp.where(s < EXTRA, NJ_BASE + 1, NJ_BASE)
    core_base = c * (QC * CH)

    def stage(j, slot):
        eb = core_base + (s + NS * j) * CH
        pltpu.async_copy(kt_hbm.at[:, pl.ds(eb, CH)], kt_v.at[slot], sem_in)
        pltpu.async_copy(lsrc_hbm.at[pl.ds(eb, CH)], ls_v.at[slot], sem_in)
        pltpu.async_copy(dst_hbm.at[pl.ds(eb, CH)], dst_v.at[slot], sem_in)

    stage(0, 0)
    iota16 = lax.iota(jnp.int32, LANES)

    z16 = jnp.zeros((LANES,), jnp.float32)

    def zb(i, _):
        zb_v[i, :] = z16
        return 0
    lax.fori_loop(0, ROWS_PER_TILE, zb, 0)

    def zc(i, _):
        zc_v[pl.ds(i * LANES, LANES)] = z16
        return 0
    lax.fori_loop(0, ROWS_PER_TILE // LANES, zc, 0)

    def o16(i, _):
        ones_v[pl.ds(i * LANES, LANES)] = jnp.ones((LANES,), jnp.float32)
        return 0
    lax.fori_loop(0, CH // LANES, o16, 0)

    pltpu.sync_copy(zb_v, acc_sh.at[pl.ds(row0, ROWS_PER_TILE)])
    pltpu.sync_copy(zc_v, cnt_sh.at[pl.ds(row0, ROWS_PER_TILE)])
    plsc.subcore_barrier()

    def lb(j, _):
        slot = lax.rem(j, 3)
        nxt = lax.rem(j + 1, 3)

        @pl.when(j >= 2)
        def _():
            pltpu.make_async_copy(msg_v.at[0], acc_sh.at[pl.ds(0, CH)],
                                  sem_sc).wait()
            pltpu.make_async_copy(ones_v, cnt_sh.at[pl.ds(0, CH)],
                                  sem_sc).wait()

        pltpu.make_async_copy(kt_hbm.at[:, pl.ds(0, CH)], kt_v.at[slot],
                              sem_in).wait()
        pltpu.make_async_copy(lsrc_hbm.at[pl.ds(0, CH)], ls_v.at[slot],
                              sem_in).wait()
        pltpu.make_async_copy(dst_hbm.at[pl.ds(0, CH)], dst_v.at[slot],
                              sem_in).wait()
        stage(jnp.minimum(j + 1, nj - 1), nxt)
        sl3 = jnp.full((LANES,), slot, jnp.int32)
        for comp in range(D_LIFT):
            c3 = jnp.full((LANES,), comp, jnp.int32)
            for gi in range(CH // LANES):
                ids = gi * LANES + iota16
                kv = kt_v[slot, comp, pl.ds(gi * LANES, LANES)]
                lg = plsc.load_gather(ls_v, [sl3, ids, c3])
                plsc.store_scatter(msg_v, [sl3, ids, c3], kv * lg)
        pltpu.async_copy(msg_v.at[slot], acc_sh.at[dst_v.at[slot]],
                         sem_sc, add=True)
        pltpu.async_copy(ones_v, cnt_sh.at[dst_v.at[slot]],
                         sem_sc, add=True)
        return 0

    lax.fori_loop(0, nj, lb, 0)
    pltpu.make_async_copy(kt_hbm.at[:, pl.ds(0, CH)], kt_v.at[0],
                          sem_in).wait()
    pltpu.make_async_copy(lsrc_hbm.at[pl.ds(0, CH)], ls_v.at[0],
                          sem_in).wait()
    pltpu.make_async_copy(dst_hbm.at[pl.ds(0, CH)], dst_v.at[0],
                          sem_in).wait()
    for _ in range(2):
        pltpu.make_async_copy(msg_v.at[0], acc_sh.at[pl.ds(0, CH)],
                              sem_sc).wait()
        pltpu.make_async_copy(ones_v, cnt_sh.at[pl.ds(0, CH)],
                              sem_sc).wait()
    plsc.subcore_barrier()

    pltpu.sync_copy(acc_sh.at[pl.ds(row0, ROWS_PER_TILE)],
                    part_hbm.at[c, pl.ds(row0, ROWS_PER_TILE)])
    pltpu.sync_copy(cnt_sh.at[pl.ds(row0, ROWS_PER_TILE)], cb_v)

    def rep(i, _):
        crep_v[pl.ds(i * LANES, LANES)] = plsc.load_gather(
            cb_v, [jnp.full((LANES,), i, jnp.int32)])
        return 0
    lax.fori_loop(0, ROWS_PER_TILE, rep, 0)
    pltpu.sync_copy(crep_v,
                    cntp_hbm.at[c, pl.ds(row0 * D_LIFT,
                                         ROWS_PER_TILE * D_LIFT)])


def _scatter(kt, lsrc, edge_dst):
    mesh = plsc.VectorSubcoreMesh(core_axis_name="c", subcore_axis_name="s",
                                  num_cores=NC, num_subcores=NS)
    return pl.kernel(
        _scatter_body,
        out_type=(jax.ShapeDtypeStruct((NC, N_LATENT, D_LIFT), jnp.float32),
                  jax.ShapeDtypeStruct((NC, N_LATENT * D_LIFT), jnp.float32)),
        mesh=mesh,
        compiler_params=pltpu.CompilerParams(
            needs_layout_passes=False, use_tc_tiling_on_sc=False),
        scratch_types=[
            pltpu.VMEM_SHARED((N_LATENT, D_LIFT), jnp.float32),
            pltpu.VMEM_SHARED((N_LATENT,), jnp.float32),
            pltpu.VMEM((3, D_LIFT, CH), jnp.float32),
            pltpu.VMEM((3, CH, D_LIFT), jnp.float32),
            pltpu.VMEM((3, CH, D_LIFT), jnp.float32),
            pltpu.VMEM((3, CH), jnp.int32),
            pltpu.VMEM((CH,), jnp.float32),
            pltpu.VMEM((ROWS_PER_TILE, D_LIFT), jnp.float32),
            pltpu.VMEM((ROWS_PER_TILE,), jnp.float32),
            pltpu.VMEM((ROWS_PER_TILE,), jnp.float32),
            pltpu.VMEM((ROWS_PER_TILE * D_LIFT,), jnp.float32),
            pltpu.SemaphoreType.DMA,
            pltpu.SemaphoreType.DMA,
        ],
    )(kt, lsrc, edge_dst)


def _combine_body(*refs):
    part_refs = refs[:-1]
    out_ref = refs[-1]
    p = part_refs[0][0] + part_refs[0][1]
    cnt = part_refs[1][0] + part_refs[1][1]
    for i in range(2, len(part_refs), 2):
        p = p + part_refs[i][0] + part_refs[i][1]
        cnt = cnt + part_refs[i + 1][0] + part_refs[i + 1][1]
    out_ref[...] = p / jnp.maximum(cnt, 1.0)


def _combine(*parts):
    return pl.pallas_call(
        _combine_body,
        out_shape=jax.ShapeDtypeStruct((N_LATENT, D_LIFT), jnp.float32),
    )(*parts)


def kernel(phys_pos, phys_feat, batch_idx_phys, edge_src, edge_dst,
           latent_tokens_pos, latent_tokens_batch_idx,
           W_lift, b_lift, W0, b0, W1, b1, W2, b2, W3, b3):
    lifted = _lift(phys_feat, W_lift, b_lift)
    parts = []
    for hidx in range(NHALF):
        sl = slice(hidx * HE, (hidx + 1) * HE)
        kin_h, lsrc_h = _gather(latent_tokens_pos, phys_pos, lifted,
                                edge_src[sl], edge_dst[sl])
        kt_h = _mlp(kin_h, W0, b0, W1, b1, W2, b2, W3, b3)
        part_h, cnt_h = _scatter(kt_h, lsrc_h, edge_dst[sl])
        parts += [part_h, cnt_h.reshape(NC, N_LATENT, D_LIFT)]
    return _combine(*parts)
